# baseline (speedup 1.0000x reference)
import functools

import jax
import jax.numpy as jnp
import numpy as np
from jax import lax
from jax.experimental import pallas as pl
from jax.experimental.pallas import tpu as pltpu
from jax.experimental.pallas import tpu_sc as plsc

NC = 2
NS = 16
LANES = 16
CHUNK = 128
NBUF = 2


def _interleave_perm(width):
    perm = np.empty((width,), dtype=np.int32)
    for g in range(width // 32):
        for t in range(16):
            perm[32 * g + 2 * t] = 32 * g + t
            perm[32 * g + 2 * t + 1] = 32 * g + 16 + t
    return perm


def kernel(s_idx, r_idx, o_idx, ent, rel, norm_w):
    B = s_idx.shape[0]
    D = ent.shape[1]
    n_workers = NC * NS
    per_w = B // n_workers
    n_chunks = per_w // CHUNK
    n_slices = D // LANES

    fused = jnp.concatenate([rel, norm_w], axis=1)
    fused = fused[:, _interleave_perm(2 * D)].astype(jnp.bfloat16)
    fused = lax.bitcast_convert_type(fused.reshape(rel.shape[0], D, 2),
                                     jnp.int32)

    eidx = jnp.stack([s_idx.reshape(n_workers, n_chunks, CHUNK),
                      o_idx.reshape(n_workers, n_chunks, CHUNK)],
                     axis=2).reshape(n_workers, n_chunks, 2 * CHUNK)

    mesh = plsc.VectorSubcoreMesh(core_axis_name="c", subcore_axis_name="s")

    @functools.partial(
        pl.kernel,
        mesh=mesh,
        out_type=jax.ShapeDtypeStruct((B,), jnp.float32),
        scratch_types=[
            pltpu.VMEM((n_chunks, 2 * CHUNK), jnp.int32),
            pltpu.VMEM((per_w,), jnp.int32),
            pltpu.VMEM((per_w,), jnp.float32),
        ] + [
            t
            for _ in range(NBUF)
            for t in (pltpu.VMEM((2 * CHUNK, D), jnp.float32),
                      pltpu.VMEM((CHUNK, D), jnp.int32),
                      pltpu.SemaphoreType.DMA)
        ],
    )
    def transh(eidx_hbm, r_hbm, ent_hbm, fused_hbm, out_hbm,
               eidx_v, ridx_v, out_v, *bufrefs):
        wid = lax.axis_index("s") * NC + lax.axis_index("c")
        base = wid * per_w

        bufs = [tuple(bufrefs[3 * b:3 * b + 3]) for b in range(NBUF)]
        sem0 = bufs[0][2]

        cps = [
            pltpu.async_copy(eidx_hbm.at[wid], eidx_v, sem0),
            pltpu.async_copy(r_hbm.at[pl.ds(base, per_w)], ridx_v, sem0),
        ]
        for cp in cps:
            cp.wait()

        def fire(chunk, b):
            esio_b, fu_b, sem = bufs[b]
            pltpu.async_copy(ent_hbm.at[eidx_v.at[chunk, pl.ds(0, CHUNK)]],
                             esio_b.at[pl.ds(0, CHUNK)], sem)
            pltpu.async_copy(ent_hbm.at[eidx_v.at[chunk, pl.ds(CHUNK, CHUNK)]],
                             esio_b.at[pl.ds(CHUNK, CHUNK)], sem)
            pltpu.async_copy(
                fused_hbm.at[ridx_v.at[pl.ds(chunk * CHUNK, CHUNK)]], fu_b, sem)

        def drain(b):
            esio_b, fu_b, sem = bufs[b]
            pltpu.make_async_copy(ent_hbm.at[eidx_v.at[0, pl.ds(0, CHUNK)]],
                                  esio_b.at[pl.ds(0, CHUNK)], sem).wait()
            pltpu.make_async_copy(ent_hbm.at[eidx_v.at[0, pl.ds(0, CHUNK)]],
                                  esio_b.at[pl.ds(CHUNK, CHUNK)], sem).wait()
            pltpu.make_async_copy(
                fused_hbm.at[ridx_v.at[pl.ds(0, CHUNK)]], fu_b, sem).wait()

        lane_ids = lax.iota(jnp.int32, LANES)
        perms = [lane_ids ^ s for s in (8, 4, 2, 1)]

        def unpack_bf16_pair(ref, i, word_col):
            x = ref[i, pl.ds(word_col, LANES)]
            even = lax.bitcast_convert_type(x << 16, jnp.float32)
            odd = lax.bitcast_convert_type(x & jnp.int32(-65536), jnp.float32)
            return even, odd

        def splat_sum(x):
            for p in perms:
                x = x + x.at[p].get(mode="promise_in_bounds")
            return x

        def compute(chunk, b):
            esio_b, fu_b, _ = bufs[b]

            def group_body(g, _):
                scores = jnp.zeros((LANES,), jnp.float32)
                for k in range(LANES):
                    i = g * LANES + k
                    d_sl = []
                    n_sl = []
                    acc_dn = jnp.zeros((LANES,), jnp.float32)
                    acc_nn = jnp.zeros((LANES,), jnp.float32)
                    for h in range(n_slices // 2):
                        na, nb = unpack_bf16_pair(fu_b, i, D // 2 + h * LANES)
                        n_sl.append(na)
                        n_sl.append(nb)
                    for j in range(n_slices):
                        sl = pl.ds(j * LANES, LANES)
                        d = esio_b[i, sl] - esio_b[CHUNK + i, sl]
                        d_sl.append(d)
                        nv = n_sl[j]
                        acc_dn = acc_dn + d * nv
                        acc_nn = acc_nn + nv * nv
                    c_v = splat_sum(acc_dn) / splat_sum(acc_nn)
                    acc_abs = jnp.zeros((LANES,), jnp.float32)
                    for h in range(n_slices // 2):
                        ra, rb = unpack_bf16_pair(fu_b, i, h * LANES)
                        for j, rv in ((2 * h, ra), (2 * h + 1, rb)):
                            diff = d_sl[j] + rv - c_v * n_sl[j]
                            acc_abs = acc_abs + jnp.abs(diff)
                    scores = jnp.where(lane_ids == k, splat_sum(acc_abs), scores)
                out_v[pl.ds(chunk * CHUNK + g * LANES, LANES)] = scores
                return 0

            lax.fori_loop(0, CHUNK // LANES, group_body, 0)

        for b in range(NBUF):
            fire(b, b)

        def ring_body(p, _):
            for b in range(NBUF):
                chunk = NBUF * p + b
                drain(b)
                compute(chunk, b)

                @pl.when(p < n_chunks // NBUF - 1)
                def _refire():
                    fire(chunk + NBUF, b)
            return 0

        lax.fori_loop(0, n_chunks // NBUF, ring_body, 0)

        pltpu.sync_copy(out_v, out_hbm.at[pl.ds(base, per_w)])

    return transh(eidx, r_idx, ent, fused)


# speedup vs baseline: 1.0381x; 1.0381x over previous
import functools

import jax
import jax.numpy as jnp
import numpy as np
from jax import lax
from jax.experimental import pallas as pl
from jax.experimental.pallas import tpu as pltpu
from jax.experimental.pallas import tpu_sc as plsc

NC = 2
NS = 16
LANES = 16
CHUNK = 128
NBUF = 2
GROUP_UNROLL = 1


def _interleave_perm(width):
    perm = np.empty((width,), dtype=np.int32)
    for g in range(width // 32):
        for t in range(16):
            perm[32 * g + 2 * t] = 32 * g + t
            perm[32 * g + 2 * t + 1] = 32 * g + 16 + t
    return perm


def kernel(s_idx, r_idx, o_idx, ent, rel, norm_w):
    B = s_idx.shape[0]
    D = ent.shape[1]
    n_workers = NC * NS
    per_w = B // n_workers
    n_chunks = per_w // CHUNK
    n_slices = D // LANES

    fused = jnp.concatenate([rel, norm_w], axis=1)
    fused = fused[:, _interleave_perm(2 * D)].astype(jnp.bfloat16)
    fused = lax.bitcast_convert_type(fused.reshape(rel.shape[0], D, 2),
                                     jnp.int32)

    mesh = plsc.VectorSubcoreMesh(core_axis_name="c", subcore_axis_name="s")

    @functools.partial(
        pl.kernel,
        mesh=mesh,
        out_type=jax.ShapeDtypeStruct((B,), jnp.float32),
        scratch_types=[
            pltpu.VMEM((per_w,), jnp.int32),
            pltpu.VMEM((per_w,), jnp.int32),
            pltpu.VMEM((per_w,), jnp.int32),
            pltpu.VMEM((per_w,), jnp.float32),
        ] + [
            t
            for _ in range(NBUF)
            for t in (pltpu.VMEM((CHUNK, D), jnp.float32),
                      pltpu.VMEM((CHUNK, D), jnp.float32),
                      pltpu.VMEM((CHUNK, D), jnp.int32),
                      pltpu.SemaphoreType.DMA)
        ],
    )
    def transh(s_hbm, r_hbm, o_hbm, ent_hbm, fused_hbm, out_hbm,
               sidx_v, ridx_v, oidx_v, out_v, *bufrefs):
        wid = lax.axis_index("s") * NC + lax.axis_index("c")
        base = wid * per_w

        bufs = [tuple(bufrefs[4 * b:4 * b + 4]) for b in range(NBUF)]
        sem0 = bufs[0][3]

        cps = [
            pltpu.async_copy(s_hbm.at[pl.ds(base, per_w)], sidx_v, sem0),
            pltpu.async_copy(o_hbm.at[pl.ds(base, per_w)], oidx_v, sem0),
            pltpu.async_copy(r_hbm.at[pl.ds(base, per_w)], ridx_v, sem0),
        ]
        for cp in cps:
            cp.wait()

        def fire(chunk, b):
            es_b, eo_b, fu_b, sem = bufs[b]
            lo = chunk * CHUNK
            pltpu.async_copy(ent_hbm.at[sidx_v.at[pl.ds(lo, CHUNK)]], es_b, sem)
            pltpu.async_copy(ent_hbm.at[oidx_v.at[pl.ds(lo, CHUNK)]], eo_b, sem)
            pltpu.async_copy(fused_hbm.at[ridx_v.at[pl.ds(lo, CHUNK)]], fu_b, sem)

        def drain(b):
            es_b, eo_b, fu_b, sem = bufs[b]
            pltpu.make_async_copy(ent_hbm.at[sidx_v.at[pl.ds(0, CHUNK)]], es_b, sem).wait()
            pltpu.make_async_copy(ent_hbm.at[oidx_v.at[pl.ds(0, CHUNK)]], eo_b, sem).wait()
            pltpu.make_async_copy(fused_hbm.at[ridx_v.at[pl.ds(0, CHUNK)]], fu_b, sem).wait()

        lane_ids = lax.iota(jnp.int32, LANES)
        perms = [lane_ids ^ s for s in (8, 4, 2, 1)]

        def unpack_bf16_pair(ref, i, word_col):
            x = ref[i, pl.ds(word_col, LANES)]
            even = lax.bitcast_convert_type(x << 16, jnp.float32)
            odd = lax.bitcast_convert_type(x & jnp.int32(-65536), jnp.float32)
            return even, odd

        def splat_sum(x):
            for p in perms:
                x = x + x.at[p].get(mode="promise_in_bounds")
            return x

        def compute(chunk, b):
            es_b, eo_b, fu_b, _ = bufs[b]

            def group_body(g, _):
                scores = jnp.zeros((LANES,), jnp.float32)
                for k in range(LANES):
                    i = g * LANES + k
                    d_sl = []
                    n_sl = []
                    acc_dn = jnp.zeros((LANES,), jnp.float32)
                    acc_nn = jnp.zeros((LANES,), jnp.float32)
                    for h in range(n_slices // 2):
                        na, nb = unpack_bf16_pair(fu_b, i, D // 2 + h * LANES)
                        n_sl.append(na)
                        n_sl.append(nb)
                    for j in range(n_slices):
                        sl = pl.ds(j * LANES, LANES)
                        d = es_b[i, sl] - eo_b[i, sl]
                        d_sl.append(d)
                        nv = n_sl[j]
                        acc_dn = acc_dn + d * nv
                        acc_nn = acc_nn + nv * nv
                    c_v = splat_sum(acc_dn) / splat_sum(acc_nn)
                    acc_abs = jnp.zeros((LANES,), jnp.float32)
                    for h in range(n_slices // 2):
                        ra, rb = unpack_bf16_pair(fu_b, i, h * LANES)
                        for j, rv in ((2 * h, ra), (2 * h + 1, rb)):
                            diff = d_sl[j] + rv - c_v * n_sl[j]
                            acc_abs = acc_abs + jnp.abs(diff)
                    scores = jnp.where(lane_ids == k, splat_sum(acc_abs), scores)
                out_v[pl.ds(chunk * CHUNK + g * LANES, LANES)] = scores
                return 0

            lax.fori_loop(0, CHUNK // LANES, group_body, 0,
                          unroll=GROUP_UNROLL)

        for b in range(NBUF):
            fire(b, b)

        def ring_body(p, _):
            for b in range(NBUF):
                chunk = NBUF * p + b
                drain(b)
                compute(chunk, b)

                @pl.when(p < n_chunks // NBUF - 1)
                def _refire():
                    fire(chunk + NBUF, b)
            return 0

        lax.fori_loop(0, n_chunks // NBUF, ring_body, 0)

        pltpu.sync_copy(out_v, out_hbm.at[pl.ds(base, per_w)])

    return transh(s_idx, r_idx, o_idx, ent, fused)
